# baseline (speedup 1.0000x reference)
import functools

import jax
import jax.numpy as jnp
from jax import lax
from jax.experimental import pallas as pl
from jax.experimental.pallas import tpu as pltpu
from jax.experimental.pallas import tpu_sc as plsc

DIM = 64
GRAN = 2740
N_DAY = 366
NC = 2
NS = 16
L = 16
N_E = 1000000
IDXCH = 128


def _sincos(x):
    x2 = x * x
    s = x * (1.0 + x2 * (-1.0 / 6.0 + x2 * (1.0 / 120.0 + x2 * (-1.0 / 5040.0))))
    c = 1.0 + x2 * (-0.5 + x2 * (1.0 / 24.0 + x2 * (-1.0 / 720.0 + x2 * (1.0 / 40320.0))))
    return s, c


def _tero_body(h_hbm, t_hbm, r_hbm, d_hbm, ert_hbm, eit_hbm, rrt_hbm, rit_hbm,
               ertl_hbm, eitl_hbm, rrtl_hbm, ritl_hbm,
               time_hbm, out_hbm,
               h_v, t_v, r_v, d_v,
               her_v, hei_v, ter_v, tei_v, rr_v, ri_v, tt_v, acc_v, tail_v,
               col_s, lsem, gsem):
    cid = lax.axis_index("c")
    sid = lax.axis_index("s")
    bpw = h_v.shape[0]
    base = sid * bpw

    pltpu.sync_copy(h_hbm.at[pl.ds(base, bpw)], h_v)
    pltpu.sync_copy(t_hbm.at[pl.ds(base, bpw)], t_v)
    pltpu.sync_copy(r_hbm.at[pl.ds(base, bpw)], r_v)
    pltpu.sync_copy(d_hbm.at[pl.ds(base, bpw)], d_v)

    def _tt_body(i, _):
        pltpu.sync_copy(time_hbm.at[i], tt_v.at[i])
        return 0
    lax.fori_loop(0, N_DAY, _tt_body, 0)

    def _div_body(i, _):
        x = d_v[pl.ds(i * L, L)].astype(jnp.float32)
        d_v[pl.ds(i * L, L)] = (x / jnp.float32(GRAN)).astype(jnp.int32)
        return 0
    lax.fori_loop(0, bpw // L, _div_body, 0, unroll=4)

    def _zero_body(i, _):
        acc_v[pl.ds(i * L, L)] = jnp.zeros((L,), jnp.float32)
        return 0
    lax.fori_loop(0, bpw // L, _zero_body, 0, unroll=4)

    nfeat = DIM // NC

    SH = (N_E // NS) // 128 * 128
    TAIL_OFF = SH * NS
    TAIL = N_E - TAIL_OFF

    def _load_col(tbl_hbm, tail_hbm, c):
        off = pl.multiple_of(sid * SH, 128)
        cp = pltpu.async_copy(
            tbl_hbm.at[c, pl.ds(off, SH)], col_s.at[pl.ds(off, SH)], lsem)

        @pl.when(sid == NS - 1)
        def _tail():
            pltpu.sync_copy(tail_hbm.at[pl.ds(c * TAIL, TAIL)], tail_v)
            pltpu.sync_copy(tail_v, col_s.at[pl.ds(TAIL_OFF, TAIL)])

        cp.wait()
        plsc.subcore_barrier()

    def _gather(idx_v, dst_v):
        cps = [
            pltpu.async_copy(
                col_s.at[idx_v.at[pl.ds(k * IDXCH, IDXCH)]],
                dst_v.at[pl.ds(k * IDXCH, IDXCH)],
                gsem,
            )
            for k in range(bpw // IDXCH)
        ]
        for cp in cps:
            cp.wait()

    def _feat_body(cc, _):
        c = cid * nfeat + cc

        _load_col(ert_hbm, ertl_hbm, c)
        _gather(h_v, her_v)
        _gather(t_v, ter_v)
        plsc.subcore_barrier()

        _load_col(eit_hbm, eitl_hbm, c)
        _gather(h_v, hei_v)
        _gather(t_v, tei_v)
        plsc.subcore_barrier()

        _load_col(rrt_hbm, rrtl_hbm, c)
        _gather(r_v, rr_v)
        plsc.subcore_barrier()

        _load_col(rit_hbm, ritl_hbm, c)
        _gather(r_v, ri_v)
        plsc.subcore_barrier()

        cvec = jnp.full((L,), c, jnp.int32)

        def _grp_body(g, _):
            sl = pl.ds(g * L, L)
            tv = plsc.load_gather(tt_v, [d_v[sl], cvec])
            s, co = _sincos(tv)
            her = her_v[sl]
            hei = hei_v[sl]
            ter = ter_v[sl]
            tei = tei_v[sl]
            treal = (her - ter) * co - (hei - tei) * s + rr_v[sl]
            timg = (her + ter) * s + (hei + tei) * co + ri_v[sl]
            acc_v[sl] = acc_v[sl] + jnp.abs(treal) + jnp.abs(timg)
            return 0

        lax.fori_loop(0, bpw // L, _grp_body, 0)
        return 0

    lax.fori_loop(0, nfeat, _feat_body, 0)
    pltpu.sync_copy(acc_v, out_hbm.at[cid, pl.ds(base, bpw)])


@jax.jit
def kernel(X, emb_E_real, emb_E_img, emb_R_real, emb_R_img, emb_Time):
    B = X.shape[0]
    bpw = B // NS

    mesh = plsc.VectorSubcoreMesh(core_axis_name="c", subcore_axis_name="s")
    run = pl.kernel(
        _tero_body,
        out_type=jax.ShapeDtypeStruct((NC, B), jnp.float32),
        mesh=mesh,
        compiler_params=pltpu.CompilerParams(needs_layout_passes=False),
        scratch_types=[
            pltpu.VMEM((bpw,), jnp.int32),
            pltpu.VMEM((bpw,), jnp.int32),
            pltpu.VMEM((bpw,), jnp.int32),
            pltpu.VMEM((bpw,), jnp.int32),
            pltpu.VMEM((bpw,), jnp.float32),
            pltpu.VMEM((bpw,), jnp.float32),
            pltpu.VMEM((bpw,), jnp.float32),
            pltpu.VMEM((bpw,), jnp.float32),
            pltpu.VMEM((bpw,), jnp.float32),
            pltpu.VMEM((bpw,), jnp.float32),
            pltpu.VMEM((N_DAY, DIM), jnp.float32),
            pltpu.VMEM((bpw,), jnp.float32),
            pltpu.VMEM((N_E - (N_E // NS) // 128 * 128 * NS,), jnp.float32),
            pltpu.VMEM_SHARED((N_E,), jnp.float32),
            pltpu.SemaphoreType.DMA,
            pltpu.SemaphoreType.DMA,
        ],
    )
    tail0 = (N_E // NS) // 128 * 128 * NS
    tails = [t[tail0:, :].T.reshape(-1)
             for t in (emb_E_real, emb_E_img, emb_R_real, emb_R_img)]
    partial_scores = run(X[:, 0], X[:, 1], X[:, 2], X[:, 3],
                         emb_E_real.T, emb_E_img.T, emb_R_real.T,
                         emb_R_img.T, *tails, emb_Time)
    return partial_scores[0] + partial_scores[1]


# speedup vs baseline: 1.0064x; 1.0064x over previous
import functools

import jax
import jax.numpy as jnp
from jax import lax
from jax.experimental import pallas as pl
from jax.experimental.pallas import tpu as pltpu
from jax.experimental.pallas import tpu_sc as plsc

DIM = 64
GRAN = 2740
N_DAY = 366
NC = 2
NS = 16
L = 16
N_E = 1000000
IDXCH = 128


def _sincos(x):
    x2 = x * x
    s = x * (1.0 + x2 * (-1.0 / 6.0 + x2 * (1.0 / 120.0 + x2 * (-1.0 / 5040.0))))
    c = 1.0 + x2 * (-0.5 + x2 * (1.0 / 24.0 + x2 * (-1.0 / 720.0 + x2 * (1.0 / 40320.0))))
    return s, c


def _tero_body(h_hbm, t_hbm, r_hbm, d_hbm, ert_hbm, eit_hbm, rrt_hbm, rit_hbm,
               ertl_hbm, eitl_hbm, rrtl_hbm, ritl_hbm,
               time_hbm, out_hbm,
               h_v, t_v, r_v, d_v,
               her_v, hei_v, ter_v, tei_v, rr_v, ri_v, tt_v, acc_v, tail_v,
               col_s, lsem, gsem):
    cid = lax.axis_index("c")
    sid = lax.axis_index("s")
    bpw = h_v.shape[0]
    base = sid * bpw

    pltpu.sync_copy(h_hbm.at[pl.ds(base, bpw)], h_v)
    pltpu.sync_copy(t_hbm.at[pl.ds(base, bpw)], t_v)
    pltpu.sync_copy(r_hbm.at[pl.ds(base, bpw)], r_v)
    pltpu.sync_copy(d_hbm.at[pl.ds(base, bpw)], d_v)

    def _tt_body(i, _):
        pltpu.sync_copy(time_hbm.at[i], tt_v.at[i])
        return 0
    lax.fori_loop(0, N_DAY, _tt_body, 0)

    def _div_body(i, _):
        x = d_v[pl.ds(i * L, L)].astype(jnp.float32)
        d_v[pl.ds(i * L, L)] = (x / jnp.float32(GRAN)).astype(jnp.int32)
        return 0
    lax.fori_loop(0, bpw // L, _div_body, 0, unroll=4)

    def _zero_body(i, _):
        acc_v[pl.ds(i * L, L)] = jnp.zeros((L,), jnp.float32)
        return 0
    lax.fori_loop(0, bpw // L, _zero_body, 0, unroll=4)

    nfeat = DIM // NC

    SH = (N_E // NS) // 128 * 128
    TAIL_OFF = SH * NS
    TAIL = N_E - TAIL_OFF

    def _load_col(tbl_hbm, tail_hbm, c):
        off = pl.multiple_of(sid * SH, 128)
        cp = pltpu.async_copy(
            tbl_hbm.at[c, pl.ds(off, SH)], col_s.at[pl.ds(off, SH)], lsem)

        @pl.when(sid == NS - 1)
        def _tail():
            pltpu.sync_copy(tail_hbm.at[pl.ds(c * TAIL, TAIL)], tail_v)
            pltpu.sync_copy(tail_v, col_s.at[pl.ds(TAIL_OFF, TAIL)])

        cp.wait()
        plsc.subcore_barrier()

    def _gather(*pairs):
        cps = [
            pltpu.async_copy(
                col_s.at[idx_v.at[pl.ds(k * IDXCH, IDXCH)]],
                dst_v.at[pl.ds(k * IDXCH, IDXCH)],
                gsem,
            )
            for idx_v, dst_v in pairs
            for k in range(bpw // IDXCH)
        ]
        for cp in cps:
            cp.wait()

    def _feat_body(cc, _):
        c = cid * nfeat + cc

        _load_col(ert_hbm, ertl_hbm, c)
        _gather((h_v, her_v), (t_v, ter_v))
        plsc.subcore_barrier()

        _load_col(eit_hbm, eitl_hbm, c)
        _gather((h_v, hei_v), (t_v, tei_v))
        plsc.subcore_barrier()

        _load_col(rrt_hbm, rrtl_hbm, c)
        _gather((r_v, rr_v))
        plsc.subcore_barrier()

        _load_col(rit_hbm, ritl_hbm, c)
        _gather((r_v, ri_v))
        plsc.subcore_barrier()

        cvec = jnp.full((L,), c, jnp.int32)

        def _grp_body(g, _):
            sl = pl.ds(g * L, L)
            tv = plsc.load_gather(tt_v, [d_v[sl], cvec])
            s, co = _sincos(tv)
            her = her_v[sl]
            hei = hei_v[sl]
            ter = ter_v[sl]
            tei = tei_v[sl]
            treal = (her - ter) * co - (hei - tei) * s + rr_v[sl]
            timg = (her + ter) * s + (hei + tei) * co + ri_v[sl]
            acc_v[sl] = acc_v[sl] + jnp.abs(treal) + jnp.abs(timg)
            return 0

        lax.fori_loop(0, bpw // L, _grp_body, 0)
        return 0

    lax.fori_loop(0, nfeat, _feat_body, 0)
    pltpu.sync_copy(acc_v, out_hbm.at[cid, pl.ds(base, bpw)])


@jax.jit
def kernel(X, emb_E_real, emb_E_img, emb_R_real, emb_R_img, emb_Time):
    B = X.shape[0]
    bpw = B // NS

    mesh = plsc.VectorSubcoreMesh(core_axis_name="c", subcore_axis_name="s")
    run = pl.kernel(
        _tero_body,
        out_type=jax.ShapeDtypeStruct((NC, B), jnp.float32),
        mesh=mesh,
        compiler_params=pltpu.CompilerParams(needs_layout_passes=False),
        scratch_types=[
            pltpu.VMEM((bpw,), jnp.int32),
            pltpu.VMEM((bpw,), jnp.int32),
            pltpu.VMEM((bpw,), jnp.int32),
            pltpu.VMEM((bpw,), jnp.int32),
            pltpu.VMEM((bpw,), jnp.float32),
            pltpu.VMEM((bpw,), jnp.float32),
            pltpu.VMEM((bpw,), jnp.float32),
            pltpu.VMEM((bpw,), jnp.float32),
            pltpu.VMEM((bpw,), jnp.float32),
            pltpu.VMEM((bpw,), jnp.float32),
            pltpu.VMEM((N_DAY, DIM), jnp.float32),
            pltpu.VMEM((bpw,), jnp.float32),
            pltpu.VMEM((N_E - (N_E // NS) // 128 * 128 * NS,), jnp.float32),
            pltpu.VMEM_SHARED((N_E,), jnp.float32),
            pltpu.SemaphoreType.DMA,
            pltpu.SemaphoreType.DMA,
        ],
    )
    tail0 = (N_E // NS) // 128 * 128 * NS
    tails = [t[tail0:, :].T.reshape(-1)
             for t in (emb_E_real, emb_E_img, emb_R_real, emb_R_img)]
    partial_scores = run(X[:, 0], X[:, 1], X[:, 2], X[:, 3],
                         emb_E_real.T, emb_E_img.T, emb_R_real.T,
                         emb_R_img.T, *tails, emb_Time)
    return partial_scores[0] + partial_scores[1]
